# baseline (speedup 1.0000x reference)
import jax
import jax.numpy as jnp
from jax import lax
from jax.experimental import pallas as pl
from jax.experimental.pallas import tpu as pltpu
from jax.experimental.pallas import tpu_sc as plsc

N_NODES = 10000
N_PAD = 10112
ROWS_PER_TILE = N_PAD // 16
N_EDGES = 320000
CHUNK = 64
EDGE_ROWS = 5120
ROWS_PER_WORKER = EDGE_ROWS // 32

NC, NS = 2, 16
NW = NC * NS


def _zero_fill(buf, n_rows, cols):
    zeros16 = jnp.zeros((16,), jnp.float32)

    @pl.loop(0, n_rows * (cols // 16))
    def _(i):
        r = i // (cols // 16)
        c = (i % (cols // 16)) * 16
        buf[r, pl.ds(c, 16)] = zeros16


def _make_segsum(d_feat, with_cnt):
    mesh = plsc.VectorSubcoreMesh(core_axis_name="c", subcore_axis_name="s",
                                  num_cores=NC, num_subcores=NS)
    out_type = [jax.ShapeDtypeStruct((NC, N_PAD, d_feat), jnp.float32)]
    if with_cnt:
        out_type.append(jax.ShapeDtypeStruct((NW * N_PAD,), jnp.float32))
    n_stage = 4
    stg = ROWS_PER_WORKER // n_stage
    scratch = [
        pltpu.VMEM_SHARED((N_PAD, d_feat), jnp.float32),
        pltpu.VMEM((stg, CHUNK), jnp.int32),
        pltpu.VMEM((stg, CHUNK), jnp.int32),
        pltpu.VMEM((CHUNK, d_feat), jnp.float32),
        pltpu.VMEM((CHUNK, d_feat), jnp.float32),
        pltpu.VMEM((CHUNK, d_feat), jnp.float32),
        pltpu.SemaphoreType.DMA,
        pltpu.SemaphoreType.DMA,
        pltpu.SemaphoreType.DMA,
        pltpu.SemaphoreType.DMA,
        pltpu.SemaphoreType.DMA,
        pltpu.SemaphoreType.DMA,
    ]
    if with_cnt:
        scratch.append(pltpu.VMEM((N_PAD,), jnp.float32))

    def body(feat, src2d, dst2d, *rest):
        if with_cnt:
            (out, cout, acc, srcbuf, dstbuf, r0, r1, r2, gs0, gs1, gs2,
             ss0, ss1, ss2, cnt_local) = rest
        else:
            (out, acc, srcbuf, dstbuf, r0, r1, r2, gs0, gs1, gs2,
             ss0, ss1, ss2) = rest
            cout = cnt_local = None
        r = (r0, r1, r2)
        gs = (gs0, gs1, gs2)
        ss = (ss0, ss1, ss2)
        rows0 = r0
        c = lax.axis_index("c")
        s = lax.axis_index("s")
        wid = s * NC + c

        scope = jax.named_scope
        with scope("zeroing"):
            _zero_fill(rows0, CHUNK, d_feat)
        if with_cnt:
            zeros16 = jnp.zeros((16,), jnp.float32)

            @pl.loop(0, N_PAD // 16)
            def _(i):
                cnt_local[pl.ds(i * 16, 16)] = zeros16

        with scope("acc_zero"):
            base = s * ROWS_PER_TILE
            full, rem = ROWS_PER_TILE // CHUNK, ROWS_PER_TILE % CHUNK
            for k in range(full):
                pltpu.sync_copy(rows0, acc.at[pl.ds(base + k * CHUNK, CHUNK)])
            if rem:
                pltpu.sync_copy(rows0.at[pl.ds(0, rem)],
                                acc.at[pl.ds(base + full * CHUNK, rem)])
            plsc.subcore_barrier()

        ebase = wid * ROWS_PER_WORKER
        ones16 = jnp.ones((16,), jnp.float32)

        def g(t, b):
            pltpu.async_copy(feat.at[srcbuf.at[t]], r[b], gs[b])

        def wg(b):
            pltpu.make_async_copy(feat.at[srcbuf.at[0]], r[b], gs[b]).wait()

        def sc(t, b):
            pltpu.async_copy(r[b], acc.at[dstbuf.at[t]], ss[b], add=True)

        def ws(b):
            pltpu.make_async_copy(r[b], acc.at[dstbuf.at[0]], ss[b]).wait()

        def counts(t):
            if with_cnt:
                for k in range(CHUNK // 16):
                    idx = dstbuf[t, pl.ds(k * 16, 16)]
                    plsc.addupdate_scatter(cnt_local, [idx], ones16)

        for st in range(n_stage):
          with scope(f"edges{st}"):
            sb = ebase + st * stg
            pltpu.sync_copy(src2d.at[pl.ds(sb, stg)], srcbuf)
            pltpu.sync_copy(dst2d.at[pl.ds(sb, stg)], dstbuf)

            g(0, 0)
            g(1, 1)
            wg(0)
            sc(0, 0)
            counts(0)
            g(2, 2)

            @pl.loop(0, (stg - 4) // 3)
            def _(i):
                t = 3 * i + 1
                wg(1); sc(t, 1); counts(t); ws(0); g(t + 2, 0)
                wg(2); sc(t + 1, 2); counts(t + 1); ws(1); g(t + 3, 1)
                wg(0); sc(t + 2, 0); counts(t + 2); ws(2); g(t + 4, 2)

            tl = stg - 3
            wg(1); sc(tl, 1); counts(tl); ws(0); g(tl + 2, 0)
            wg(2); sc(tl + 1, 2); counts(tl + 1); ws(1)
            wg(0); sc(tl + 2, 0); counts(tl + 2); ws(2); ws(0)

        with scope("writeback"):
            plsc.subcore_barrier()

            pltpu.sync_copy(acc.at[pl.ds(base, ROWS_PER_TILE)],
                            out.at[c, pl.ds(base, ROWS_PER_TILE)])
            if with_cnt:
                pltpu.sync_copy(cnt_local, cout.at[pl.ds(wid * N_PAD, N_PAD)])

    return pl.kernel(body, out_type=out_type, mesh=mesh, scratch_types=scratch,
                     compiler_params=pltpu.CompilerParams(needs_layout_passes=False))


_segsum_l1 = _make_segsum(128, with_cnt=True)
_segsum_l2 = _make_segsum(128, with_cnt=False)

_ROW_BLK = 1000


def _tc1_body(s1_ref, cnt_ref, x_ref, wl1_ref, wr1_ref, b1_ref, wl2_ref,
              wr2_ref, b2_ref, p_ref, q_ref):
    tot = jnp.maximum(jnp.sum(cnt_ref[...], axis=1), 1.0)
    agg = (s1_ref[0] + s1_ref[1]) / tot[:, None]
    h = agg @ wl1_ref[...] + x_ref[...] @ wr1_ref[...] + b1_ref[...]
    h = jnp.maximum(h, 0.0)
    p_ref[...] = jnp.concatenate(
        [h @ wl2_ref[...], jnp.zeros((h.shape[0], 64), jnp.float32)], axis=1)
    q_ref[...] = h @ wr2_ref[...] + b2_ref[...]


def _tc2_body(s2_ref, cnt_ref, q_ref, o_ref):
    tot = jnp.maximum(jnp.sum(cnt_ref[...], axis=1), 1.0)
    z = (s2_ref[0] + s2_ref[1]) / tot[:, None] + q_ref[...]
    m = jnp.max(z, axis=1, keepdims=True)
    e = jnp.exp(z - m)
    o_ref[...] = (z - m) - jnp.log(jnp.sum(e, axis=1, keepdims=True))


def _tc1(s1, cnt_t, x, Wl1, Wr1, b1, Wl2, Wr2, b2):
    n_blk = N_NODES // _ROW_BLK
    blk = lambda shape, imap: pl.BlockSpec(shape, imap)
    return pl.pallas_call(
        _tc1_body,
        grid=(n_blk,),
        in_specs=[
            blk((2, _ROW_BLK, 128), lambda i: (0, i, 0)),
            blk((_ROW_BLK, NW), lambda i: (i, 0)),
            blk((_ROW_BLK, 128), lambda i: (i, 0)),
            blk((128, 256), lambda i: (0, 0)),
            blk((128, 256), lambda i: (0, 0)),
            blk((1, 256), lambda i: (0, 0)),
            blk((256, 64), lambda i: (0, 0)),
            blk((256, 64), lambda i: (0, 0)),
            blk((1, 64), lambda i: (0, 0)),
        ],
        out_specs=[
            blk((_ROW_BLK, 128), lambda i: (i, 0)),
            blk((_ROW_BLK, 64), lambda i: (i, 0)),
        ],
        out_shape=[
            jax.ShapeDtypeStruct((N_NODES, 128), jnp.float32),
            jax.ShapeDtypeStruct((N_NODES, 64), jnp.float32),
        ],
    )(s1, cnt_t, x, Wl1, Wr1, b1.reshape(1, 256), Wl2, Wr2, b2.reshape(1, 64))


def _tc2(s2, cnt_t, q):
    n_blk = N_NODES // _ROW_BLK
    blk = lambda shape, imap: pl.BlockSpec(shape, imap)
    return pl.pallas_call(
        _tc2_body,
        grid=(n_blk,),
        in_specs=[
            blk((2, _ROW_BLK, 64), lambda i: (0, i, 0)),
            blk((_ROW_BLK, NW), lambda i: (i, 0)),
            blk((_ROW_BLK, 64), lambda i: (i, 0)),
        ],
        out_specs=blk((_ROW_BLK, 64), lambda i: (i, 0)),
        out_shape=jax.ShapeDtypeStruct((N_NODES, 64), jnp.float32),
    )(s2, cnt_t, q)


def kernel(x, edge_index, Wl1, Wr1, b1, Wl2, Wr2, b2):
    src = edge_index[0].astype(jnp.int32)
    dst = edge_index[1].astype(jnp.int32)
    pad = EDGE_ROWS * CHUNK - N_EDGES
    src2d = jnp.concatenate(
        [src, jnp.zeros((pad,), jnp.int32)]).reshape(EDGE_ROWS, CHUNK)
    dst2d = jnp.concatenate(
        [dst, jnp.full((pad,), N_NODES, jnp.int32)]).reshape(EDGE_ROWS, CHUNK)

    s1p, cntp = _segsum_l1(x, src2d, dst2d)
    cnt_t = jnp.transpose(cntp.reshape(NW, N_PAD)[:, :N_NODES])
    p, q = _tc1(s1p[:, :N_NODES], cnt_t, x, Wl1, Wr1, b1, Wl2, Wr2, b2)
    (s2p,) = _segsum_l2(p, src2d, dst2d)
    return _tc2(s2p[:, :N_NODES, :64], cnt_t, q)


# speedup vs baseline: 3.1557x; 3.1557x over previous
import jax
import jax.numpy as jnp
from jax import lax
from jax.experimental import pallas as pl
from jax.experimental.pallas import tpu as pltpu
from jax.experimental.pallas import tpu_sc as plsc

N_NODES = 10000
N_PAD = 10112
ROWS_PER_TILE = N_PAD // 16
N_EDGES = 320000
CHUNK = 64
EDGE_ROWS = 5120
ROWS_PER_WORKER = EDGE_ROWS // 32

NC, NS = 2, 16
NW = NC * NS


def _zero_fill(buf, n_rows, cols):
    zeros16 = jnp.zeros((16,), jnp.float32)

    @pl.loop(0, n_rows * (cols // 16))
    def _(i):
        r = i // (cols // 16)
        c = (i % (cols // 16)) * 16
        buf[r, pl.ds(c, 16)] = zeros16


def _make_segsum(d_feat, with_cnt):
    mesh = plsc.VectorSubcoreMesh(core_axis_name="c", subcore_axis_name="s",
                                  num_cores=NC, num_subcores=NS)
    out_type = [jax.ShapeDtypeStruct((NC, N_PAD, d_feat), jnp.float32)]
    if with_cnt:
        out_type.append(jax.ShapeDtypeStruct((NW * N_PAD,), jnp.float32))
    n_stage = 4
    stg = ROWS_PER_WORKER // n_stage
    scratch = [
        pltpu.VMEM_SHARED((N_PAD, d_feat), jnp.float32),
        pltpu.VMEM((stg, CHUNK), jnp.int32),
        pltpu.VMEM((stg, CHUNK), jnp.int32),
        pltpu.VMEM((CHUNK, d_feat), jnp.float32),
        pltpu.VMEM((CHUNK, d_feat), jnp.float32),
        pltpu.VMEM((CHUNK, d_feat), jnp.float32),
        pltpu.SemaphoreType.DMA,
        pltpu.SemaphoreType.DMA,
        pltpu.SemaphoreType.DMA,
        pltpu.SemaphoreType.DMA,
        pltpu.SemaphoreType.DMA,
        pltpu.SemaphoreType.DMA,
    ]
    if with_cnt:
        scratch.append(pltpu.VMEM((N_PAD,), jnp.float32))

    def body(feat, src2d, dst2d, *rest):
        if with_cnt:
            (out, cout, acc, srcbuf, dstbuf, r0, r1, r2, gs0, gs1, gs2,
             ss0, ss1, ss2, cnt_local) = rest
        else:
            (out, acc, srcbuf, dstbuf, r0, r1, r2, gs0, gs1, gs2,
             ss0, ss1, ss2) = rest
            cout = cnt_local = None
        r = (r0, r1, r2)
        gs = (gs0, gs1, gs2)
        ss = (ss0, ss1, ss2)
        rows0 = r0
        c = lax.axis_index("c")
        s = lax.axis_index("s")
        wid = s * NC + c

        scope = jax.named_scope
        with scope("zeroing"):
            _zero_fill(rows0, CHUNK, d_feat)
        if with_cnt:
            zeros16 = jnp.zeros((16,), jnp.float32)

            @pl.loop(0, N_PAD // 16)
            def _(i):
                cnt_local[pl.ds(i * 16, 16)] = zeros16

        with scope("acc_zero"):
            base = s * ROWS_PER_TILE
            full, rem = ROWS_PER_TILE // CHUNK, ROWS_PER_TILE % CHUNK
            for k in range(full):
                pltpu.sync_copy(rows0, acc.at[pl.ds(base + k * CHUNK, CHUNK)])
            if rem:
                pltpu.sync_copy(rows0.at[pl.ds(0, rem)],
                                acc.at[pl.ds(base + full * CHUNK, rem)])
            plsc.subcore_barrier()

        ebase = wid * ROWS_PER_WORKER
        ones16 = jnp.ones((16,), jnp.float32)

        def g(t, b):
            pltpu.async_copy(feat.at[srcbuf.at[t]], r[b], gs[b])

        def wg(b):
            pltpu.make_async_copy(feat.at[srcbuf.at[0]], r[b], gs[b]).wait()

        def sc(t, b):
            pltpu.async_copy(r[b], acc.at[dstbuf.at[t]], ss[b], add=True)

        def ws(b):
            pltpu.make_async_copy(r[b], acc.at[dstbuf.at[0]], ss[b]).wait()

        def counts(t):
            if with_cnt:
                for k in range(CHUNK // 16):
                    idx = dstbuf[t, pl.ds(k * 16, 16)]
                    plsc.addupdate_scatter(cnt_local, [idx], ones16)

        for st in range(n_stage):
          with scope(f"edges{st}"):
            sb = ebase + st * stg
            pltpu.sync_copy(src2d.at[pl.ds(sb, stg)], srcbuf)
            pltpu.sync_copy(dst2d.at[pl.ds(sb, stg)], dstbuf)

            g(0, 0)
            g(1, 1)
            wg(0)
            sc(0, 0)
            counts(0)
            g(2, 2)

            @pl.loop(0, (stg - 4) // 3)
            def _(i):
                t = 3 * i + 1
                wg(1); sc(t, 1); counts(t); ws(0); g(t + 2, 0)
                wg(2); sc(t + 1, 2); counts(t + 1); ws(1); g(t + 3, 1)
                wg(0); sc(t + 2, 0); counts(t + 2); ws(2); g(t + 4, 2)

            tl = stg - 3
            wg(1); sc(tl, 1); counts(tl); ws(0); g(tl + 2, 0)
            wg(2); sc(tl + 1, 2); counts(tl + 1); ws(1)
            wg(0); sc(tl + 2, 0); counts(tl + 2); ws(2); ws(0)

        with scope("writeback"):
            plsc.subcore_barrier()

            pltpu.sync_copy(acc.at[pl.ds(base, ROWS_PER_TILE)],
                            out.at[c, pl.ds(base, ROWS_PER_TILE)])
            if with_cnt:
                pltpu.sync_copy(cnt_local, cout.at[pl.ds(wid * N_PAD, N_PAD)])

    return pl.kernel(body, out_type=out_type, mesh=mesh, scratch_types=scratch,
                     compiler_params=pltpu.CompilerParams(needs_layout_passes=False))


_segsum_l1 = _make_segsum(128, with_cnt=True)
_segsum_l2 = _make_segsum(128, with_cnt=False)

_ROW_BLK = 1000


def _tc1_body(s1_ref, cnt_ref, x_ref, wl1_ref, wr1_ref, b1_ref, wl2_ref,
              wr2_ref, b2_ref, p_ref, q_ref):
    tot = jnp.maximum(jnp.sum(cnt_ref[...], axis=1), 1.0)
    agg = (s1_ref[0] + s1_ref[1]) / tot[:, None]
    h = agg @ wl1_ref[...] + x_ref[...] @ wr1_ref[...] + b1_ref[...]
    h = jnp.maximum(h, 0.0)
    p_ref[...] = jnp.concatenate(
        [h @ wl2_ref[...], jnp.zeros((h.shape[0], 64), jnp.float32)], axis=1)
    q_ref[...] = h @ wr2_ref[...] + b2_ref[...]


def _tc2_body(s2_ref, cnt_ref, q_ref, o_ref):
    tot = jnp.maximum(jnp.sum(cnt_ref[...], axis=1), 1.0)
    z = (s2_ref[0] + s2_ref[1]) / tot[:, None] + q_ref[...]
    m = jnp.max(z, axis=1, keepdims=True)
    e = jnp.exp(z - m)
    o_ref[...] = (z - m) - jnp.log(jnp.sum(e, axis=1, keepdims=True))


def _tc1(s1, cnt_t, x, Wl1, Wr1, b1, Wl2, Wr2, b2):
    n_blk = N_NODES // _ROW_BLK
    blk = lambda shape, imap: pl.BlockSpec(shape, imap)
    return pl.pallas_call(
        _tc1_body,
        grid=(n_blk,),
        in_specs=[
            blk((2, _ROW_BLK, 128), lambda i: (0, i, 0)),
            blk((_ROW_BLK, NW), lambda i: (i, 0)),
            blk((_ROW_BLK, 128), lambda i: (i, 0)),
            blk((128, 256), lambda i: (0, 0)),
            blk((128, 256), lambda i: (0, 0)),
            blk((1, 256), lambda i: (0, 0)),
            blk((256, 64), lambda i: (0, 0)),
            blk((256, 64), lambda i: (0, 0)),
            blk((1, 64), lambda i: (0, 0)),
        ],
        out_specs=[
            blk((_ROW_BLK, 128), lambda i: (i, 0)),
            blk((_ROW_BLK, 64), lambda i: (i, 0)),
        ],
        out_shape=[
            jax.ShapeDtypeStruct((N_NODES, 128), jnp.float32),
            jax.ShapeDtypeStruct((N_NODES, 64), jnp.float32),
        ],
    )(s1, cnt_t, x, Wl1, Wr1, b1.reshape(1, 256), Wl2, Wr2, b2.reshape(1, 64))


def _tc2(s2, cnt_t, q):
    n_blk = N_NODES // _ROW_BLK
    blk = lambda shape, imap: pl.BlockSpec(shape, imap)
    return pl.pallas_call(
        _tc2_body,
        grid=(n_blk,),
        in_specs=[
            blk((2, _ROW_BLK, 64), lambda i: (0, i, 0)),
            blk((_ROW_BLK, NW), lambda i: (i, 0)),
            blk((_ROW_BLK, 64), lambda i: (i, 0)),
        ],
        out_specs=blk((_ROW_BLK, 64), lambda i: (i, 0)),
        out_shape=jax.ShapeDtypeStruct((N_NODES, 64), jnp.float32),
    )(s2, cnt_t, q)


def kernel(x, edge_index, Wl1, Wr1, b1, Wl2, Wr2, b2):
    src = edge_index[0].astype(jnp.int32)
    dst = edge_index[1].astype(jnp.int32)
    pad = EDGE_ROWS * CHUNK - N_EDGES
    pad_iota = jnp.arange(pad, dtype=jnp.int32)
    src2d = jnp.concatenate(
        [src, pad_iota % N_NODES]).reshape(EDGE_ROWS, CHUNK)
    dst2d = jnp.concatenate(
        [dst, N_NODES + pad_iota % (N_PAD - N_NODES)]).reshape(EDGE_ROWS, CHUNK)

    s1p, cntp = _segsum_l1(x, src2d, dst2d)
    cnt_t = jnp.transpose(cntp.reshape(NW, N_PAD)[:, :N_NODES])
    p, q = _tc1(s1p[:, :N_NODES], cnt_t, x, Wl1, Wr1, b1, Wl2, Wr2, b2)
    (s2p,) = _segsum_l2(p, src2d, dst2d)
    return _tc2(s2p[:, :N_NODES, :64], cnt_t, q)


# speedup vs baseline: 3.3506x; 1.0617x over previous
import jax
import jax.numpy as jnp
from jax import lax
from jax.experimental import pallas as pl
from jax.experimental.pallas import tpu as pltpu
from jax.experimental.pallas import tpu_sc as plsc

N_NODES = 10000
N_PAD = 10112
ROWS_PER_TILE = N_PAD // 16
N_EDGES = 320000
CHUNK = 64
EDGE_ROWS = 5120
ROWS_PER_WORKER = EDGE_ROWS // 32

NC, NS = 2, 16
NW = NC * NS


def _zero_fill(buf, n_rows, cols):
    zeros16 = jnp.zeros((16,), jnp.float32)

    @pl.loop(0, n_rows * (cols // 16))
    def _(i):
        r = i // (cols // 16)
        c = (i % (cols // 16)) * 16
        buf[r, pl.ds(c, 16)] = zeros16


def _make_segsum(d_feat, with_cnt, tc_tiling=True):
    mesh = plsc.VectorSubcoreMesh(core_axis_name="c", subcore_axis_name="s",
                                  num_cores=NC, num_subcores=NS)
    out_type = [jax.ShapeDtypeStruct((NC, N_PAD, d_feat), jnp.float32)]
    if with_cnt:
        out_type.append(jax.ShapeDtypeStruct((NW * N_PAD,), jnp.float32))
    n_stage = 4
    stg = ROWS_PER_WORKER // n_stage
    scratch = [
        pltpu.VMEM_SHARED((N_PAD, d_feat), jnp.float32),
        pltpu.VMEM((stg, CHUNK), jnp.int32),
        pltpu.VMEM((stg, CHUNK), jnp.int32),
        pltpu.VMEM((CHUNK, d_feat), jnp.float32),
        pltpu.VMEM((CHUNK, d_feat), jnp.float32),
        pltpu.VMEM((CHUNK, d_feat), jnp.float32),
        pltpu.SemaphoreType.DMA,
        pltpu.SemaphoreType.DMA,
        pltpu.SemaphoreType.DMA,
        pltpu.SemaphoreType.DMA,
        pltpu.SemaphoreType.DMA,
        pltpu.SemaphoreType.DMA,
    ]
    if with_cnt:
        scratch.append(pltpu.VMEM((N_PAD,), jnp.float32))

    def body(feat, src2d, dst2d, *rest):
        if with_cnt:
            (out, cout, acc, srcbuf, dstbuf, r0, r1, r2, gs0, gs1, gs2,
             ss0, ss1, ss2, cnt_local) = rest
        else:
            (out, acc, srcbuf, dstbuf, r0, r1, r2, gs0, gs1, gs2,
             ss0, ss1, ss2) = rest
            cout = cnt_local = None
        r = (r0, r1, r2)
        gs = (gs0, gs1, gs2)
        ss = (ss0, ss1, ss2)
        rows0 = r0
        c = lax.axis_index("c")
        s = lax.axis_index("s")
        wid = s * NC + c

        scope = jax.named_scope
        with scope("zeroing"):
            _zero_fill(rows0, CHUNK, d_feat)
        if with_cnt:
            zeros16 = jnp.zeros((16,), jnp.float32)

            @pl.loop(0, N_PAD // 16)
            def _(i):
                cnt_local[pl.ds(i * 16, 16)] = zeros16

        with scope("acc_zero"):
            base = s * ROWS_PER_TILE
            full, rem = ROWS_PER_TILE // CHUNK, ROWS_PER_TILE % CHUNK
            for k in range(full):
                pltpu.sync_copy(rows0, acc.at[pl.ds(base + k * CHUNK, CHUNK)])
            if rem:
                pltpu.sync_copy(rows0.at[pl.ds(0, rem)],
                                acc.at[pl.ds(base + full * CHUNK, rem)])
            plsc.subcore_barrier()

        ebase = wid * ROWS_PER_WORKER
        ones16 = jnp.ones((16,), jnp.float32)

        def g(t, b):
            pltpu.async_copy(feat.at[srcbuf.at[t]], r[b], gs[b])

        def wg(b):
            pltpu.make_async_copy(feat.at[srcbuf.at[0]], r[b], gs[b]).wait()

        def sc(t, b):
            pltpu.async_copy(r[b], acc.at[dstbuf.at[t]], ss[b], add=True)

        def ws(b):
            pltpu.make_async_copy(r[b], acc.at[dstbuf.at[0]], ss[b]).wait()

        def counts(t):
            if with_cnt:
                for k in range(CHUNK // 16):
                    idx = dstbuf[t, pl.ds(k * 16, 16)]
                    plsc.addupdate_scatter(cnt_local, [idx], ones16)

        for st in range(n_stage):
          with scope(f"edges{st}"):
            sb = ebase + st * stg
            pltpu.sync_copy(src2d.at[pl.ds(sb, stg)], srcbuf)
            pltpu.sync_copy(dst2d.at[pl.ds(sb, stg)], dstbuf)

            g(0, 0)
            g(1, 1)
            wg(0)
            sc(0, 0)
            counts(0)
            g(2, 2)

            @pl.loop(0, (stg - 4) // 3)
            def _(i):
                t = 3 * i + 1
                wg(1); sc(t, 1); counts(t); ws(0); g(t + 2, 0)
                wg(2); sc(t + 1, 2); counts(t + 1); ws(1); g(t + 3, 1)
                wg(0); sc(t + 2, 0); counts(t + 2); ws(2); g(t + 4, 2)

            tl = stg - 3
            wg(1); sc(tl, 1); counts(tl); ws(0); g(tl + 2, 0)
            wg(2); sc(tl + 1, 2); counts(tl + 1); ws(1)
            wg(0); sc(tl + 2, 0); counts(tl + 2); ws(2); ws(0)

        with scope("writeback"):
            plsc.subcore_barrier()

            pltpu.sync_copy(acc.at[pl.ds(base, ROWS_PER_TILE)],
                            out.at[c, pl.ds(base, ROWS_PER_TILE)])
            if with_cnt:
                pltpu.sync_copy(cnt_local, cout.at[pl.ds(wid * N_PAD, N_PAD)])

    return pl.kernel(body, out_type=out_type, mesh=mesh, scratch_types=scratch,
                     compiler_params=pltpu.CompilerParams(
                         needs_layout_passes=False,
                         use_tc_tiling_on_sc=tc_tiling))


_segsum_l1 = _make_segsum(128, with_cnt=True)
_segsum_l2 = _make_segsum(64, with_cnt=False, tc_tiling=False)

_ROW_BLK = 1000


def _tc1_body(s1_ref, cnt_ref, x_ref, wl1_ref, wr1_ref, b1_ref, wl2_ref,
              wr2_ref, b2_ref, p_ref, q_ref):
    tot = jnp.maximum(jnp.sum(cnt_ref[...], axis=1), 1.0)
    agg = (s1_ref[0] + s1_ref[1]) / tot[:, None]
    h = agg @ wl1_ref[...] + x_ref[...] @ wr1_ref[...] + b1_ref[...]
    h = jnp.maximum(h, 0.0)
    p_ref[...] = h @ wl2_ref[...]
    q_ref[...] = h @ wr2_ref[...] + b2_ref[...]


def _tc2_body(s2_ref, cnt_ref, q_ref, o_ref):
    tot = jnp.maximum(jnp.sum(cnt_ref[...], axis=1), 1.0)
    z = (s2_ref[0] + s2_ref[1]) / tot[:, None] + q_ref[...]
    m = jnp.max(z, axis=1, keepdims=True)
    e = jnp.exp(z - m)
    o_ref[...] = (z - m) - jnp.log(jnp.sum(e, axis=1, keepdims=True))


def _tc1(s1, cnt_t, x, Wl1, Wr1, b1, Wl2, Wr2, b2):
    n_blk = N_NODES // _ROW_BLK
    blk = lambda shape, imap: pl.BlockSpec(shape, imap)
    return pl.pallas_call(
        _tc1_body,
        grid=(n_blk,),
        in_specs=[
            blk((2, _ROW_BLK, 128), lambda i: (0, i, 0)),
            blk((_ROW_BLK, NW), lambda i: (i, 0)),
            blk((_ROW_BLK, 128), lambda i: (i, 0)),
            blk((128, 256), lambda i: (0, 0)),
            blk((128, 256), lambda i: (0, 0)),
            blk((1, 256), lambda i: (0, 0)),
            blk((256, 64), lambda i: (0, 0)),
            blk((256, 64), lambda i: (0, 0)),
            blk((1, 64), lambda i: (0, 0)),
        ],
        out_specs=[
            blk((_ROW_BLK, 64), lambda i: (i, 0)),
            blk((_ROW_BLK, 64), lambda i: (i, 0)),
        ],
        out_shape=[
            jax.ShapeDtypeStruct((N_NODES, 64), jnp.float32),
            jax.ShapeDtypeStruct((N_NODES, 64), jnp.float32),
        ],
    )(s1, cnt_t, x, Wl1, Wr1, b1.reshape(1, 256), Wl2, Wr2, b2.reshape(1, 64))


def _tc2(s2, cnt_t, q):
    n_blk = N_NODES // _ROW_BLK
    blk = lambda shape, imap: pl.BlockSpec(shape, imap)
    return pl.pallas_call(
        _tc2_body,
        grid=(n_blk,),
        in_specs=[
            blk((2, _ROW_BLK, 64), lambda i: (0, i, 0)),
            blk((_ROW_BLK, NW), lambda i: (i, 0)),
            blk((_ROW_BLK, 64), lambda i: (i, 0)),
        ],
        out_specs=blk((_ROW_BLK, 64), lambda i: (i, 0)),
        out_shape=jax.ShapeDtypeStruct((N_NODES, 64), jnp.float32),
    )(s2, cnt_t, q)


def kernel(x, edge_index, Wl1, Wr1, b1, Wl2, Wr2, b2):
    src = edge_index[0].astype(jnp.int32)
    dst = edge_index[1].astype(jnp.int32)
    pad = EDGE_ROWS * CHUNK - N_EDGES
    pad_iota = jnp.arange(pad, dtype=jnp.int32)
    src2d = jnp.concatenate(
        [src, pad_iota % N_NODES]).reshape(EDGE_ROWS, CHUNK)
    dst2d = jnp.concatenate(
        [dst, N_NODES + pad_iota % (N_PAD - N_NODES)]).reshape(EDGE_ROWS, CHUNK)

    s1p, cntp = _segsum_l1(x, src2d, dst2d)
    cnt_t = jnp.transpose(cntp.reshape(NW, N_PAD)[:, :N_NODES])
    p, q = _tc1(s1p[:, :N_NODES], cnt_t, x, Wl1, Wr1, b1, Wl2, Wr2, b2)
    (s2p,) = _segsum_l2(p, src2d, dst2d)
    return _tc2(s2p[:, :N_NODES], cnt_t, q)


# speedup vs baseline: 3.5250x; 1.0521x over previous
import jax
import jax.numpy as jnp
from jax import lax
from jax.experimental import pallas as pl
from jax.experimental.pallas import tpu as pltpu
from jax.experimental.pallas import tpu_sc as plsc

N_NODES = 10000
N_PAD = 10112
ROWS_PER_TILE = N_PAD // 16
N_EDGES = 320000
CHUNK = 64
EDGE_ROWS = 5120
ROWS_PER_WORKER = EDGE_ROWS // 32

NC, NS = 2, 16
NW = NC * NS


def _zero_fill(buf, n_rows, cols):
    zeros16 = jnp.zeros((16,), jnp.float32)

    @pl.loop(0, n_rows * (cols // 16))
    def _(i):
        r = i // (cols // 16)
        c = (i % (cols // 16)) * 16
        buf[r, pl.ds(c, 16)] = zeros16


def _make_segsum(d_feat, with_cnt, tc_tiling=True):
    mesh = plsc.VectorSubcoreMesh(core_axis_name="c", subcore_axis_name="s",
                                  num_cores=NC, num_subcores=NS)
    out_type = [jax.ShapeDtypeStruct((NC, N_PAD, d_feat), jnp.float32)]
    if with_cnt:
        out_type.append(jax.ShapeDtypeStruct((NW * N_PAD,), jnp.float32))
    n_stage = 4
    stg = ROWS_PER_WORKER // n_stage
    scratch = [
        pltpu.VMEM_SHARED((N_PAD, d_feat), jnp.float32),
        pltpu.VMEM((stg, CHUNK), jnp.int32),
        pltpu.VMEM((stg, CHUNK), jnp.int32),
        pltpu.VMEM((CHUNK, d_feat), jnp.float32),
        pltpu.VMEM((CHUNK, d_feat), jnp.float32),
        pltpu.VMEM((CHUNK, d_feat), jnp.float32),
        pltpu.SemaphoreType.DMA,
        pltpu.SemaphoreType.DMA,
        pltpu.SemaphoreType.DMA,
        pltpu.SemaphoreType.DMA,
        pltpu.SemaphoreType.DMA,
        pltpu.SemaphoreType.DMA,
    ]
    if with_cnt:
        scratch.append(pltpu.VMEM((N_PAD,), jnp.float32))

    def body(feat, src2d, dst2d, *rest):
        if with_cnt:
            (out, cout, acc, srcbuf, dstbuf, r0, r1, r2, gs0, gs1, gs2,
             ss0, ss1, ss2, cnt_local) = rest
        else:
            (out, acc, srcbuf, dstbuf, r0, r1, r2, gs0, gs1, gs2,
             ss0, ss1, ss2) = rest
            cout = cnt_local = None
        r = (r0, r1, r2)
        gs = (gs0, gs1, gs2)
        ss = (ss0, ss1, ss2)
        rows0 = r0
        c = lax.axis_index("c")
        s = lax.axis_index("s")
        wid = s * NC + c

        scope = jax.named_scope
        with scope("zeroing"):
            _zero_fill(rows0, CHUNK, d_feat)
        if with_cnt:
            zeros16 = jnp.zeros((16,), jnp.float32)

            @pl.loop(0, N_PAD // 16)
            def _(i):
                cnt_local[pl.ds(i * 16, 16)] = zeros16

        with scope("acc_zero"):
            base = s * ROWS_PER_TILE
            full, rem = ROWS_PER_TILE // CHUNK, ROWS_PER_TILE % CHUNK
            for k in range(full):
                pltpu.sync_copy(rows0, acc.at[pl.ds(base + k * CHUNK, CHUNK)])
            if rem:
                pltpu.sync_copy(rows0.at[pl.ds(0, rem)],
                                acc.at[pl.ds(base + full * CHUNK, rem)])
            plsc.subcore_barrier()

        ebase = wid * ROWS_PER_WORKER
        ones16 = jnp.ones((16,), jnp.float32)

        def g(t, b):
            pltpu.async_copy(feat.at[srcbuf.at[t]], r[b], gs[b])

        def wg(b):
            pltpu.make_async_copy(feat.at[srcbuf.at[0]], r[b], gs[b]).wait()

        def sc(t, b):
            pltpu.async_copy(r[b], acc.at[dstbuf.at[t]], ss[b], add=True)

        def ws(b):
            pltpu.make_async_copy(r[b], acc.at[dstbuf.at[0]], ss[b]).wait()

        def counts(t):
            if with_cnt:
                for k in range(CHUNK // 16):
                    idx = dstbuf[t, pl.ds(k * 16, 16)]
                    plsc.addupdate_scatter(cnt_local, [idx], ones16)

        for st in range(n_stage):
          with scope(f"edges{st}"):
            sb = ebase + st * stg
            pltpu.sync_copy(src2d.at[pl.ds(sb, stg)], srcbuf)
            pltpu.sync_copy(dst2d.at[pl.ds(sb, stg)], dstbuf)

            g(0, 0)
            g(1, 1)
            wg(0)
            sc(0, 0)
            counts(0)
            g(2, 2)

            @pl.loop(0, (stg - 4) // 3)
            def _(i):
                t = 3 * i + 1
                wg(1); sc(t, 1); counts(t); ws(0); g(t + 2, 0)
                wg(2); sc(t + 1, 2); counts(t + 1); ws(1); g(t + 3, 1)
                wg(0); sc(t + 2, 0); counts(t + 2); ws(2); g(t + 4, 2)

            tl = stg - 3
            wg(1); sc(tl, 1); counts(tl); ws(0); g(tl + 2, 0)
            wg(2); sc(tl + 1, 2); counts(tl + 1); ws(1)
            wg(0); sc(tl + 2, 0); counts(tl + 2); ws(2); ws(0)

        with scope("writeback"):
            plsc.subcore_barrier()

            pltpu.sync_copy(acc.at[pl.ds(base, ROWS_PER_TILE)],
                            out.at[c, pl.ds(base, ROWS_PER_TILE)])
            if with_cnt:
                pltpu.sync_copy(cnt_local, cout.at[pl.ds(wid * N_PAD, N_PAD)])

    return pl.kernel(body, out_type=out_type, mesh=mesh, scratch_types=scratch,
                     compiler_params=pltpu.CompilerParams(
                         needs_layout_passes=False,
                         use_tc_tiling_on_sc=tc_tiling))


_segsum_l1 = _make_segsum(128, with_cnt=True)
_segsum_l2 = _make_segsum(64, with_cnt=False, tc_tiling=False)

_ROW_BLK = 1000


def _tc0_body(x_ref, wr1_ref, b1_ref, xr_ref):
    xr_ref[...] = x_ref[...] @ wr1_ref[...] + b1_ref[...]


def _tc1_body(s1_ref, cnt_ref, xr_ref, wl1_ref, wl2_ref, wr2_ref, b2_ref,
              p_ref, q_ref):
    tot = jnp.maximum(jnp.sum(cnt_ref[...], axis=1), 1.0)
    agg = (s1_ref[0] + s1_ref[1]) / tot[:, None]
    h = jnp.maximum(agg @ wl1_ref[...] + xr_ref[...], 0.0)
    p_ref[...] = h @ wl2_ref[...]
    q_ref[...] = h @ wr2_ref[...] + b2_ref[...]


def _tc2_body(s2_ref, cnt_ref, q_ref, o_ref):
    tot = jnp.maximum(jnp.sum(cnt_ref[...], axis=1), 1.0)
    z = (s2_ref[0] + s2_ref[1]) / tot[:, None] + q_ref[...]
    m = jnp.max(z, axis=1, keepdims=True)
    e = jnp.exp(z - m)
    o_ref[...] = (z - m) - jnp.log(jnp.sum(e, axis=1, keepdims=True))


def _tc0(x, Wr1, b1):
    n_blk = N_NODES // _ROW_BLK
    blk = lambda shape, imap: pl.BlockSpec(shape, imap)
    return pl.pallas_call(
        _tc0_body,
        grid=(n_blk,),
        in_specs=[
            blk((_ROW_BLK, 128), lambda i: (i, 0)),
            blk((128, 256), lambda i: (0, 0)),
            blk((1, 256), lambda i: (0, 0)),
        ],
        out_specs=blk((_ROW_BLK, 256), lambda i: (i, 0)),
        out_shape=jax.ShapeDtypeStruct((N_NODES, 256), jnp.float32),
    )(x, Wr1, b1.reshape(1, 256))


def _tc1(s1p, cnt_t, xr, Wl1, Wl2, Wr2, b2):
    n_blk = N_NODES // _ROW_BLK
    blk = lambda shape, imap: pl.BlockSpec(shape, imap)
    return pl.pallas_call(
        _tc1_body,
        grid=(n_blk,),
        in_specs=[
            blk((2, _ROW_BLK, 128), lambda i: (0, i, 0)),
            blk((_ROW_BLK, NW), lambda i: (i, 0)),
            blk((_ROW_BLK, 256), lambda i: (i, 0)),
            blk((128, 256), lambda i: (0, 0)),
            blk((256, 64), lambda i: (0, 0)),
            blk((256, 64), lambda i: (0, 0)),
            blk((1, 64), lambda i: (0, 0)),
        ],
        out_specs=[
            blk((_ROW_BLK, 64), lambda i: (i, 0)),
            blk((_ROW_BLK, 64), lambda i: (i, 0)),
        ],
        out_shape=[
            jax.ShapeDtypeStruct((N_NODES, 64), jnp.float32),
            jax.ShapeDtypeStruct((N_NODES, 64), jnp.float32),
        ],
    )(s1p, cnt_t, xr, Wl1, Wl2, Wr2, b2.reshape(1, 64))


def _tc2(s2p, cnt_t, q):
    n_blk = N_NODES // _ROW_BLK
    blk = lambda shape, imap: pl.BlockSpec(shape, imap)
    return pl.pallas_call(
        _tc2_body,
        grid=(n_blk,),
        in_specs=[
            blk((2, _ROW_BLK, 64), lambda i: (0, i, 0)),
            blk((_ROW_BLK, NW), lambda i: (i, 0)),
            blk((_ROW_BLK, 64), lambda i: (i, 0)),
        ],
        out_specs=blk((_ROW_BLK, 64), lambda i: (i, 0)),
        out_shape=jax.ShapeDtypeStruct((N_NODES, 64), jnp.float32),
    )(s2p, cnt_t, q)


def kernel(x, edge_index, Wl1, Wr1, b1, Wl2, Wr2, b2):
    src = edge_index[0].astype(jnp.int32)
    dst = edge_index[1].astype(jnp.int32)
    pad = EDGE_ROWS * CHUNK - N_EDGES
    pad_iota = jnp.arange(pad, dtype=jnp.int32)
    src2d = jnp.concatenate(
        [src, pad_iota % N_NODES]).reshape(EDGE_ROWS, CHUNK)
    dst2d = jnp.concatenate(
        [dst, N_NODES + pad_iota % (N_PAD - N_NODES)]).reshape(EDGE_ROWS, CHUNK)

    xr = _tc0(x, Wr1, b1)
    s1p, cntp = _segsum_l1(x, src2d, dst2d)
    cnt_t = jnp.transpose(cntp.reshape(NW, N_PAD)[:, :N_NODES])
    p, q = _tc1(s1p, cnt_t, xr, Wl1, Wl2, Wr2, b2)
    (s2p,) = _segsum_l2(p, src2d, dst2d)
    return _tc2(s2p, cnt_t, q)


# speedup vs baseline: 3.8032x; 1.0789x over previous
import jax
import jax.numpy as jnp
from jax import lax
from jax.experimental import pallas as pl
from jax.experimental.pallas import tpu as pltpu
from jax.experimental.pallas import tpu_sc as plsc

N_NODES = 10000
N_PAD = 10112
ROWS_PER_TILE = N_PAD // 16
N_EDGES = 320000
CHUNK = 64
EDGE_ROWS = 5120
ROWS_PER_WORKER = EDGE_ROWS // 32

NC, NS = 2, 16
NW = NC * NS


def _zero_fill(buf, n_rows, cols):
    zeros16 = jnp.zeros((16,), jnp.float32)

    @pl.loop(0, n_rows * (cols // 16))
    def _(i):
        r = i // (cols // 16)
        c = (i % (cols // 16)) * 16
        buf[r, pl.ds(c, 16)] = zeros16


def _make_segsum(d_feat, with_cnt, tc_tiling=True, chunk=CHUNK, n_stage=4):
    mesh = plsc.VectorSubcoreMesh(core_axis_name="c", subcore_axis_name="s",
                                  num_cores=NC, num_subcores=NS)
    out_type = [jax.ShapeDtypeStruct((NC, N_PAD, d_feat), jnp.float32)]
    if with_cnt:
        out_type.append(jax.ShapeDtypeStruct((NW * N_PAD,), jnp.float32))
    n_chunks = (EDGE_ROWS * CHUNK) // chunk // 32
    stg = n_chunks // n_stage
    scratch = [
        pltpu.VMEM_SHARED((N_PAD, d_feat), jnp.float32),
        pltpu.VMEM((stg, chunk), jnp.int32),
        pltpu.VMEM((stg, chunk), jnp.int32),
        pltpu.VMEM((chunk, d_feat), jnp.float32),
        pltpu.VMEM((chunk, d_feat), jnp.float32),
        pltpu.VMEM((chunk, d_feat), jnp.float32),
        pltpu.SemaphoreType.DMA,
        pltpu.SemaphoreType.DMA,
        pltpu.SemaphoreType.DMA,
        pltpu.SemaphoreType.DMA,
        pltpu.SemaphoreType.DMA,
        pltpu.SemaphoreType.DMA,
    ]
    if with_cnt:
        scratch.append(pltpu.VMEM((N_PAD,), jnp.float32))

    def body(feat, src2d, dst2d, *rest):
        if with_cnt:
            (out, cout, acc, srcbuf, dstbuf, r0, r1, r2, gs0, gs1, gs2,
             ss0, ss1, ss2, cnt_local) = rest
        else:
            (out, acc, srcbuf, dstbuf, r0, r1, r2, gs0, gs1, gs2,
             ss0, ss1, ss2) = rest
            cout = cnt_local = None
        r = (r0, r1, r2)
        gs = (gs0, gs1, gs2)
        ss = (ss0, ss1, ss2)
        rows0 = r0
        c = lax.axis_index("c")
        s = lax.axis_index("s")
        wid = s * NC + c

        scope = jax.named_scope
        with scope("zeroing"):
            _zero_fill(rows0, min(chunk, 128), d_feat)
        if with_cnt:
            zeros16 = jnp.zeros((16,), jnp.float32)

            @pl.loop(0, N_PAD // 16)
            def _(i):
                cnt_local[pl.ds(i * 16, 16)] = zeros16

        with scope("acc_zero"):
            base = s * ROWS_PER_TILE
            zc = min(chunk, 128)
            full, rem = ROWS_PER_TILE // zc, ROWS_PER_TILE % zc
            for k in range(full):
                pltpu.sync_copy(rows0.at[pl.ds(0, zc)],
                                acc.at[pl.ds(base + k * zc, zc)])
            if rem:
                pltpu.sync_copy(rows0.at[pl.ds(0, rem)],
                                acc.at[pl.ds(base + full * zc, rem)])
            plsc.subcore_barrier()

        ebase = wid * n_chunks
        ones16 = jnp.ones((16,), jnp.float32)

        def g(t, b):
            pltpu.async_copy(feat.at[srcbuf.at[t]], r[b], gs[b])

        def wg(b):
            pltpu.make_async_copy(feat.at[srcbuf.at[0]], r[b], gs[b]).wait()

        def sc(t, b):
            pltpu.async_copy(r[b], acc.at[dstbuf.at[t]], ss[b], add=True)

        def ws(b):
            pltpu.make_async_copy(r[b], acc.at[dstbuf.at[0]], ss[b]).wait()

        def counts(t):
            if with_cnt:
                for k in range(chunk // 16):
                    idx = dstbuf[t, pl.ds(k * 16, 16)]
                    plsc.addupdate_scatter(cnt_local, [idx], ones16)

        for st in range(n_stage):
          with scope(f"edges{st}"):
            sb = ebase + st * stg
            pltpu.sync_copy(src2d.at[pl.ds(sb, stg)], srcbuf)
            pltpu.sync_copy(dst2d.at[pl.ds(sb, stg)], dstbuf)

            g(0, 0)
            g(1, 1)
            wg(0)
            sc(0, 0)
            counts(0)
            g(2, 2)

            @pl.loop(0, (stg - 4) // 3)
            def _(i):
                t = 3 * i + 1
                wg(1); sc(t, 1); counts(t); ws(0); g(t + 2, 0)
                wg(2); sc(t + 1, 2); counts(t + 1); ws(1); g(t + 3, 1)
                wg(0); sc(t + 2, 0); counts(t + 2); ws(2); g(t + 4, 2)

            tl = stg - 3
            wg(1); sc(tl, 1); counts(tl); ws(0); g(tl + 2, 0)
            wg(2); sc(tl + 1, 2); counts(tl + 1); ws(1)
            wg(0); sc(tl + 2, 0); counts(tl + 2); ws(2); ws(0)

        with scope("writeback"):
            plsc.subcore_barrier()

            pltpu.sync_copy(acc.at[pl.ds(base, ROWS_PER_TILE)],
                            out.at[c, pl.ds(base, ROWS_PER_TILE)])
            if with_cnt:
                pltpu.sync_copy(cnt_local, cout.at[pl.ds(wid * N_PAD, N_PAD)])

    return pl.kernel(body, out_type=out_type, mesh=mesh, scratch_types=scratch,
                     compiler_params=pltpu.CompilerParams(
                         needs_layout_passes=False,
                         use_tc_tiling_on_sc=tc_tiling))


_segsum_l1 = _make_segsum(128, with_cnt=True)
_segsum_l2 = _make_segsum(64, with_cnt=False, tc_tiling=False,
                          chunk=128, n_stage=2)

_ROW_BLK = 1000


def _tc0_body(x_ref, wr1_ref, b1_ref, xr_ref):
    xr_ref[...] = x_ref[...] @ wr1_ref[...] + b1_ref[...]


def _tc1_body(s1_ref, cnt_ref, xr_ref, wl1_ref, wl2_ref, wr2_ref, b2_ref,
              p_ref, q_ref):
    tot = jnp.maximum(jnp.sum(cnt_ref[...], axis=1), 1.0)
    agg = (s1_ref[0] + s1_ref[1]) / tot[:, None]
    h = jnp.maximum(agg @ wl1_ref[...] + xr_ref[...], 0.0)
    p_ref[...] = h @ wl2_ref[...]
    q_ref[...] = h @ wr2_ref[...] + b2_ref[...]


def _tc2_body(s2_ref, cnt_ref, q_ref, o_ref):
    tot = jnp.maximum(jnp.sum(cnt_ref[...], axis=1), 1.0)
    z = (s2_ref[0] + s2_ref[1]) / tot[:, None] + q_ref[...]
    m = jnp.max(z, axis=1, keepdims=True)
    e = jnp.exp(z - m)
    o_ref[...] = (z - m) - jnp.log(jnp.sum(e, axis=1, keepdims=True))


def _tc0(x, Wr1, b1):
    n_blk = N_NODES // _ROW_BLK
    blk = lambda shape, imap: pl.BlockSpec(shape, imap)
    return pl.pallas_call(
        _tc0_body,
        grid=(n_blk,),
        in_specs=[
            blk((_ROW_BLK, 128), lambda i: (i, 0)),
            blk((128, 256), lambda i: (0, 0)),
            blk((1, 256), lambda i: (0, 0)),
        ],
        out_specs=blk((_ROW_BLK, 256), lambda i: (i, 0)),
        out_shape=jax.ShapeDtypeStruct((N_NODES, 256), jnp.float32),
    )(x, Wr1, b1.reshape(1, 256))


def _tc1(s1p, cnt_t, xr, Wl1, Wl2, Wr2, b2):
    n_blk = N_NODES // _ROW_BLK
    blk = lambda shape, imap: pl.BlockSpec(shape, imap)
    return pl.pallas_call(
        _tc1_body,
        grid=(n_blk,),
        in_specs=[
            blk((2, _ROW_BLK, 128), lambda i: (0, i, 0)),
            blk((_ROW_BLK, NW), lambda i: (i, 0)),
            blk((_ROW_BLK, 256), lambda i: (i, 0)),
            blk((128, 256), lambda i: (0, 0)),
            blk((256, 64), lambda i: (0, 0)),
            blk((256, 64), lambda i: (0, 0)),
            blk((1, 64), lambda i: (0, 0)),
        ],
        out_specs=[
            blk((_ROW_BLK, 64), lambda i: (i, 0)),
            blk((_ROW_BLK, 64), lambda i: (i, 0)),
        ],
        out_shape=[
            jax.ShapeDtypeStruct((N_NODES, 64), jnp.float32),
            jax.ShapeDtypeStruct((N_NODES, 64), jnp.float32),
        ],
    )(s1p, cnt_t, xr, Wl1, Wl2, Wr2, b2.reshape(1, 64))


def _tc2(s2p, cnt_t, q):
    n_blk = N_NODES // _ROW_BLK
    blk = lambda shape, imap: pl.BlockSpec(shape, imap)
    return pl.pallas_call(
        _tc2_body,
        grid=(n_blk,),
        in_specs=[
            blk((2, _ROW_BLK, 64), lambda i: (0, i, 0)),
            blk((_ROW_BLK, NW), lambda i: (i, 0)),
            blk((_ROW_BLK, 64), lambda i: (i, 0)),
        ],
        out_specs=blk((_ROW_BLK, 64), lambda i: (i, 0)),
        out_shape=jax.ShapeDtypeStruct((N_NODES, 64), jnp.float32),
    )(s2p, cnt_t, q)


def kernel(x, edge_index, Wl1, Wr1, b1, Wl2, Wr2, b2):
    src = edge_index[0].astype(jnp.int32)
    dst = edge_index[1].astype(jnp.int32)
    pad = EDGE_ROWS * CHUNK - N_EDGES
    pad_iota = jnp.arange(pad, dtype=jnp.int32)
    src2d = jnp.concatenate(
        [src, pad_iota % N_NODES]).reshape(EDGE_ROWS, CHUNK)
    dst2d = jnp.concatenate(
        [dst, N_NODES + pad_iota % (N_PAD - N_NODES)]).reshape(EDGE_ROWS, CHUNK)

    xr = _tc0(x, Wr1, b1)
    s1p, cntp = _segsum_l1(x, src2d, dst2d)
    cnt_t = jnp.transpose(cntp.reshape(NW, N_PAD)[:, :N_NODES])
    p, q = _tc1(s1p, cnt_t, xr, Wl1, Wl2, Wr2, b2)
    (s2p,) = _segsum_l2(p, src2d.reshape(EDGE_ROWS // 2, 128),
                        dst2d.reshape(EDGE_ROWS // 2, 128))
    return _tc2(s2p, cnt_t, q)


# speedup vs baseline: 3.8047x; 1.0004x over previous
import jax
import jax.numpy as jnp
import numpy as np
from jax import lax
from jax.experimental import pallas as pl
from jax.experimental.pallas import tpu as pltpu
from jax.experimental.pallas import tpu_sc as plsc

N_NODES = 10000
N_PAD = 10112
ROWS_PER_TILE = N_PAD // 16
N_EDGES = 320000
CHUNK = 64
EDGE_ROWS = 5120
ROWS_PER_WORKER = EDGE_ROWS // 32

NC, NS = 2, 16
NW = NC * NS


def _zero_fill(buf, n_rows, cols):
    zeros16 = jnp.zeros((16,), jnp.float32)

    @pl.loop(0, n_rows * (cols // 16))
    def _(i):
        r = i // (cols // 16)
        c = (i % (cols // 16)) * 16
        buf[r, pl.ds(c, 16)] = zeros16


def _make_segsum(d_feat, with_cnt, tc_tiling=True, chunk=CHUNK, n_stage=4):
    mesh = plsc.VectorSubcoreMesh(core_axis_name="c", subcore_axis_name="s",
                                  num_cores=NC, num_subcores=NS)
    out_type = [jax.ShapeDtypeStruct((NC, N_PAD, d_feat), jnp.float32)]
    if with_cnt:
        out_type.append(jax.ShapeDtypeStruct((NW * N_PAD,), jnp.float32))
    n_chunks = (EDGE_ROWS * CHUNK) // chunk // 32
    stg = n_chunks // n_stage
    scratch = [
        pltpu.VMEM_SHARED((N_PAD, d_feat), jnp.float32),
        pltpu.VMEM((stg, chunk), jnp.int32),
        pltpu.VMEM((stg, chunk), jnp.int32),
        pltpu.VMEM((chunk, d_feat), jnp.float32),
        pltpu.VMEM((chunk, d_feat), jnp.float32),
        pltpu.VMEM((chunk, d_feat), jnp.float32),
        pltpu.SemaphoreType.DMA,
        pltpu.SemaphoreType.DMA,
        pltpu.SemaphoreType.DMA,
        pltpu.SemaphoreType.DMA,
        pltpu.SemaphoreType.DMA,
        pltpu.SemaphoreType.DMA,
    ]
    if with_cnt:
        scratch.append(pltpu.VMEM((N_PAD,), jnp.float32))

    def body(feat, src2d, dst2d, *rest):
        if with_cnt:
            (out, cout, acc, srcbuf, dstbuf, r0, r1, r2, gs0, gs1, gs2,
             ss0, ss1, ss2, cnt_local) = rest
        else:
            (out, acc, srcbuf, dstbuf, r0, r1, r2, gs0, gs1, gs2,
             ss0, ss1, ss2) = rest
            cout = cnt_local = None
        r = (r0, r1, r2)
        gs = (gs0, gs1, gs2)
        ss = (ss0, ss1, ss2)
        rows0 = r0
        c = lax.axis_index("c")
        s = lax.axis_index("s")
        wid = s * NC + c

        scope = jax.named_scope
        with scope("zeroing"):
            _zero_fill(rows0, min(chunk, 128), d_feat)
        if with_cnt:
            zeros16 = jnp.zeros((16,), jnp.float32)

            @pl.loop(0, N_PAD // 16)
            def _(i):
                cnt_local[pl.ds(i * 16, 16)] = zeros16

        with scope("acc_zero"):
            base = s * ROWS_PER_TILE
            zc = min(chunk, 128)
            full, rem = ROWS_PER_TILE // zc, ROWS_PER_TILE % zc
            for k in range(full):
                pltpu.sync_copy(rows0.at[pl.ds(0, zc)],
                                acc.at[pl.ds(base + k * zc, zc)])
            if rem:
                pltpu.sync_copy(rows0.at[pl.ds(0, rem)],
                                acc.at[pl.ds(base + full * zc, rem)])
            plsc.subcore_barrier()

        ebase = wid * n_chunks
        ones16 = jnp.ones((16,), jnp.float32)

        def g(t, b):
            pltpu.async_copy(feat.at[srcbuf.at[t]], r[b], gs[b])

        def wg(b):
            pltpu.make_async_copy(feat.at[srcbuf.at[0]], r[b], gs[b]).wait()

        def sc(t, b):
            pltpu.async_copy(r[b], acc.at[dstbuf.at[t]], ss[b], add=True)

        def ws(b):
            pltpu.make_async_copy(r[b], acc.at[dstbuf.at[0]], ss[b]).wait()

        def counts(t):
            if with_cnt:
                for k in range(chunk // 16):
                    idx = dstbuf[t, pl.ds(k * 16, 16)]
                    plsc.addupdate_scatter(cnt_local, [idx], ones16)

        for st in range(n_stage):
          with scope(f"edges{st}"):
            sb = ebase + st * stg
            pltpu.sync_copy(src2d.at[pl.ds(sb, stg)], srcbuf)
            pltpu.sync_copy(dst2d.at[pl.ds(sb, stg)], dstbuf)

            g(0, 0)
            g(1, 1)
            wg(0)
            sc(0, 0)
            counts(0)
            g(2, 2)

            @pl.loop(0, (stg - 4) // 3)
            def _(i):
                t = 3 * i + 1
                wg(1); sc(t, 1); counts(t); ws(0); g(t + 2, 0)
                wg(2); sc(t + 1, 2); counts(t + 1); ws(1); g(t + 3, 1)
                wg(0); sc(t + 2, 0); counts(t + 2); ws(2); g(t + 4, 2)

            tl = stg - 3
            wg(1); sc(tl, 1); counts(tl); ws(0); g(tl + 2, 0)
            wg(2); sc(tl + 1, 2); counts(tl + 1); ws(1)
            wg(0); sc(tl + 2, 0); counts(tl + 2); ws(2); ws(0)

        with scope("writeback"):
            if with_cnt:
                pltpu.sync_copy(cnt_local, cout.at[pl.ds(wid * N_PAD, N_PAD)])
            plsc.subcore_barrier()

            pltpu.sync_copy(acc.at[pl.ds(base, ROWS_PER_TILE)],
                            out.at[c, pl.ds(base, ROWS_PER_TILE)])

    return pl.kernel(body, out_type=out_type, mesh=mesh, scratch_types=scratch,
                     compiler_params=pltpu.CompilerParams(
                         needs_layout_passes=False,
                         use_tc_tiling_on_sc=tc_tiling))


_segsum_l1 = _make_segsum(128, with_cnt=True)
_segsum_l2 = _make_segsum(64, with_cnt=False, tc_tiling=False,
                          chunk=128, n_stage=2)

_ROW_BLK = 1000


def _tc0_body(x_ref, wr1_ref, b1_ref, xr_ref):
    xr_ref[...] = x_ref[...] @ wr1_ref[...] + b1_ref[...]


def _tc1_body(s1_ref, cnt_ref, xr_ref, wl1_ref, wl2_ref, wr2_ref, b2_ref,
              p_ref, q_ref):
    tot = jnp.maximum(jnp.sum(cnt_ref[...], axis=1), 1.0)
    agg = (s1_ref[0] + s1_ref[1]) / tot[:, None]
    h = jnp.maximum(agg @ wl1_ref[...] + xr_ref[...], 0.0)
    p_ref[...] = h @ wl2_ref[...]
    q_ref[...] = h @ wr2_ref[...] + b2_ref[...]


def _tc2_body(s2_ref, cnt_ref, q_ref, o_ref):
    tot = jnp.maximum(jnp.sum(cnt_ref[...], axis=1), 1.0)
    z = (s2_ref[0] + s2_ref[1]) / tot[:, None] + q_ref[...]
    m = jnp.max(z, axis=1, keepdims=True)
    e = jnp.exp(z - m)
    o_ref[...] = (z - m) - jnp.log(jnp.sum(e, axis=1, keepdims=True))


def _tc0(x, Wr1, b1):
    n_blk = N_NODES // _ROW_BLK
    blk = lambda shape, imap: pl.BlockSpec(shape, imap)
    return pl.pallas_call(
        _tc0_body,
        grid=(n_blk,),
        in_specs=[
            blk((_ROW_BLK, 128), lambda i: (i, 0)),
            blk((128, 256), lambda i: (0, 0)),
            blk((1, 256), lambda i: (0, 0)),
        ],
        out_specs=blk((_ROW_BLK, 256), lambda i: (i, 0)),
        out_shape=jax.ShapeDtypeStruct((N_NODES, 256), jnp.float32),
    )(x, Wr1, b1.reshape(1, 256))


def _tc1(s1p, cnt_t, xr, Wl1, Wl2, Wr2, b2):
    n_blk = N_NODES // _ROW_BLK
    blk = lambda shape, imap: pl.BlockSpec(shape, imap)
    return pl.pallas_call(
        _tc1_body,
        grid=(n_blk,),
        in_specs=[
            blk((2, _ROW_BLK, 128), lambda i: (0, i, 0)),
            blk((_ROW_BLK, NW), lambda i: (i, 0)),
            blk((_ROW_BLK, 256), lambda i: (i, 0)),
            blk((128, 256), lambda i: (0, 0)),
            blk((256, 64), lambda i: (0, 0)),
            blk((256, 64), lambda i: (0, 0)),
            blk((1, 64), lambda i: (0, 0)),
        ],
        out_specs=[
            blk((_ROW_BLK, 64), lambda i: (i, 0)),
            blk((_ROW_BLK, 64), lambda i: (i, 0)),
        ],
        out_shape=[
            jax.ShapeDtypeStruct((N_NODES, 64), jnp.float32),
            jax.ShapeDtypeStruct((N_NODES, 64), jnp.float32),
        ],
    )(s1p, cnt_t, xr, Wl1, Wl2, Wr2, b2.reshape(1, 64))


def _tc2(s2p, cnt_t, q):
    n_blk = N_NODES // _ROW_BLK
    blk = lambda shape, imap: pl.BlockSpec(shape, imap)
    return pl.pallas_call(
        _tc2_body,
        grid=(n_blk,),
        in_specs=[
            blk((2, _ROW_BLK, 64), lambda i: (0, i, 0)),
            blk((_ROW_BLK, NW), lambda i: (i, 0)),
            blk((_ROW_BLK, 64), lambda i: (i, 0)),
        ],
        out_specs=blk((_ROW_BLK, 64), lambda i: (i, 0)),
        out_shape=jax.ShapeDtypeStruct((N_NODES, 64), jnp.float32),
    )(s2p, cnt_t, q)


def kernel(x, edge_index, Wl1, Wr1, b1, Wl2, Wr2, b2):
    src = edge_index[0].astype(jnp.int32)
    dst = edge_index[1].astype(jnp.int32)
    pad = EDGE_ROWS * CHUNK - N_EDGES
    pad_iota = np.arange(pad, dtype=np.int32)
    src2d = jnp.concatenate(
        [src, jnp.asarray(pad_iota % N_NODES)]).reshape(EDGE_ROWS, CHUNK)
    dst2d = jnp.concatenate(
        [dst, jnp.asarray(N_NODES + pad_iota % (N_PAD - N_NODES))]
    ).reshape(EDGE_ROWS, CHUNK)

    xr = _tc0(x, Wr1, b1)
    s1p, cntp = _segsum_l1(x, src2d, dst2d)
    cnt_t = jnp.transpose(cntp.reshape(NW, N_PAD)[:, :N_NODES])
    p, q = _tc1(s1p, cnt_t, xr, Wl1, Wl2, Wr2, b2)
    (s2p,) = _segsum_l2(p, src2d.reshape(EDGE_ROWS // 2, 128),
                        dst2d.reshape(EDGE_ROWS // 2, 128))
    return _tc2(s2p, cnt_t, q)


# speedup vs baseline: 3.8454x; 1.0107x over previous
import jax
import jax.numpy as jnp
import numpy as np
from jax import lax
from jax.experimental import pallas as pl
from jax.experimental.pallas import tpu as pltpu
from jax.experimental.pallas import tpu_sc as plsc

N_NODES = 10000
N_PAD = 10112
ROWS_PER_TILE = N_PAD // 16
N_EDGES = 320000
CHUNK = 64
EDGE_ROWS = 5120
ROWS_PER_WORKER = EDGE_ROWS // 32

NC, NS = 2, 16
NW = NC * NS


def _zero_fill(buf, n_rows, cols):
    zeros16 = jnp.zeros((16,), jnp.float32)

    @pl.loop(0, n_rows * (cols // 16))
    def _(i):
        r = i // (cols // 16)
        c = (i % (cols // 16)) * 16
        buf[r, pl.ds(c, 16)] = zeros16


def _make_segsum(d_feat, with_cnt, tc_tiling=True, chunk=CHUNK, n_stage=4):
    mesh = plsc.VectorSubcoreMesh(core_axis_name="c", subcore_axis_name="s",
                                  num_cores=NC, num_subcores=NS)
    out_type = [jax.ShapeDtypeStruct((NC, N_PAD, d_feat), jnp.float32)]
    if with_cnt:
        out_type.append(jax.ShapeDtypeStruct((NW * N_PAD,), jnp.float32))
    n_chunks = (EDGE_ROWS * CHUNK) // chunk // 32
    stg = n_chunks // n_stage
    scratch = [
        pltpu.VMEM_SHARED((N_PAD, d_feat), jnp.float32),
        pltpu.VMEM((stg, chunk), jnp.int32),
        pltpu.VMEM((stg, chunk), jnp.int32),
        pltpu.VMEM((chunk, d_feat), jnp.float32),
        pltpu.VMEM((chunk, d_feat), jnp.float32),
        pltpu.VMEM((chunk, d_feat), jnp.float32),
        pltpu.SemaphoreType.DMA,
        pltpu.SemaphoreType.DMA,
        pltpu.SemaphoreType.DMA,
        pltpu.SemaphoreType.DMA,
        pltpu.SemaphoreType.DMA,
        pltpu.SemaphoreType.DMA,
    ]
    if with_cnt:
        scratch.append(pltpu.VMEM((N_PAD,), jnp.float32))

    def body(feat, src2d, dst2d, *rest):
        if with_cnt:
            (out, cout, acc, srcbuf, dstbuf, r0, r1, r2, gs0, gs1, gs2,
             ss0, ss1, ss2, cnt_local) = rest
        else:
            (out, acc, srcbuf, dstbuf, r0, r1, r2, gs0, gs1, gs2,
             ss0, ss1, ss2) = rest
            cout = cnt_local = None
        r = (r0, r1, r2)
        gs = (gs0, gs1, gs2)
        ss = (ss0, ss1, ss2)
        rows0 = r0
        c = lax.axis_index("c")
        s = lax.axis_index("s")
        wid = s * NC + c

        scope = jax.named_scope
        with scope("zeroing"):
            _zero_fill(rows0, min(chunk, 128), d_feat)
        if with_cnt:
            zeros16 = jnp.zeros((16,), jnp.float32)

            @pl.loop(0, N_PAD // 16)
            def _(i):
                cnt_local[pl.ds(i * 16, 16)] = zeros16

        with scope("acc_zero"):
            base = s * ROWS_PER_TILE
            zc = min(chunk, 128)
            full, rem = ROWS_PER_TILE // zc, ROWS_PER_TILE % zc
            for k in range(full):
                pltpu.sync_copy(rows0.at[pl.ds(0, zc)],
                                acc.at[pl.ds(base + k * zc, zc)])
            if rem:
                pltpu.sync_copy(rows0.at[pl.ds(0, rem)],
                                acc.at[pl.ds(base + full * zc, rem)])
            plsc.subcore_barrier()

        ebase = wid * n_chunks
        ones16 = jnp.ones((16,), jnp.float32)

        def g(t, b):
            pltpu.async_copy(feat.at[srcbuf.at[t]], r[b], gs[b])

        def wg(b):
            pltpu.make_async_copy(feat.at[srcbuf.at[0]], r[b], gs[b]).wait()

        def sc(t, b):
            pltpu.async_copy(r[b], acc.at[dstbuf.at[t]], ss[b], add=True)

        def ws(b):
            pltpu.make_async_copy(r[b], acc.at[dstbuf.at[0]], ss[b]).wait()

        def counts(t):
            if with_cnt:
                for k in range(chunk // 16):
                    idx = dstbuf[t, pl.ds(k * 16, 16)]
                    plsc.addupdate_scatter(cnt_local, [idx], ones16)

        for st in range(n_stage):
          with scope(f"edges{st}"):
            sb = ebase + st * stg
            pltpu.sync_copy(src2d.at[pl.ds(sb, stg)], srcbuf)
            pltpu.sync_copy(dst2d.at[pl.ds(sb, stg)], dstbuf)

            g(0, 0)
            g(1, 1)
            wg(0)
            sc(0, 0)
            counts(0)
            g(2, 2)

            @pl.loop(0, (stg - 4) // 3)
            def _(i):
                t = 3 * i + 1
                wg(1); sc(t, 1); counts(t); ws(0); g(t + 2, 0)
                wg(2); sc(t + 1, 2); counts(t + 1); ws(1); g(t + 3, 1)
                wg(0); sc(t + 2, 0); counts(t + 2); ws(2); g(t + 4, 2)

            tl = stg - 3
            wg(1); sc(tl, 1); counts(tl); ws(0); g(tl + 2, 0)
            wg(2); sc(tl + 1, 2); counts(tl + 1); ws(1)
            wg(0); sc(tl + 2, 0); counts(tl + 2); ws(2); ws(0)

        with scope("writeback"):
            if with_cnt:
                pltpu.sync_copy(cnt_local, cout.at[pl.ds(wid * N_PAD, N_PAD)])
            plsc.subcore_barrier()

            pltpu.sync_copy(acc.at[pl.ds(base, ROWS_PER_TILE)],
                            out.at[c, pl.ds(base, ROWS_PER_TILE)])

    return pl.kernel(body, out_type=out_type, mesh=mesh, scratch_types=scratch,
                     compiler_params=pltpu.CompilerParams(
                         needs_layout_passes=False,
                         use_tc_tiling_on_sc=tc_tiling))


_segsum_l1 = _make_segsum(128, with_cnt=True)
_segsum_l2 = _make_segsum(64, with_cnt=False, tc_tiling=False,
                          chunk=128, n_stage=2)

_ROW_BLK = 2000


def _tc0_body(x_ref, wr1_ref, b1_ref, xr_ref):
    xr_ref[...] = x_ref[...] @ wr1_ref[...] + b1_ref[...]


def _tc1_body(s1_ref, cnt_ref, xr_ref, wl1_ref, wl2_ref, wr2_ref, b2_ref,
              p_ref, q_ref):
    tot = jnp.maximum(jnp.sum(cnt_ref[...], axis=1), 1.0)
    agg = (s1_ref[0] + s1_ref[1]) / tot[:, None]
    h = jnp.maximum(agg @ wl1_ref[...] + xr_ref[...], 0.0)
    p_ref[...] = h @ wl2_ref[...]
    q_ref[...] = h @ wr2_ref[...] + b2_ref[...]


def _tc2_body(s2_ref, cnt_ref, q_ref, o_ref):
    tot = jnp.maximum(jnp.sum(cnt_ref[...], axis=1), 1.0)
    z = (s2_ref[0] + s2_ref[1]) / tot[:, None] + q_ref[...]
    m = jnp.max(z, axis=1, keepdims=True)
    e = jnp.exp(z - m)
    o_ref[...] = (z - m) - jnp.log(jnp.sum(e, axis=1, keepdims=True))


def _tc0(x, Wr1, b1):
    n_blk = N_NODES // _ROW_BLK
    blk = lambda shape, imap: pl.BlockSpec(shape, imap)
    return pl.pallas_call(
        _tc0_body,
        grid=(n_blk,),
        in_specs=[
            blk((_ROW_BLK, 128), lambda i: (i, 0)),
            blk((128, 256), lambda i: (0, 0)),
            blk((1, 256), lambda i: (0, 0)),
        ],
        out_specs=blk((_ROW_BLK, 256), lambda i: (i, 0)),
        out_shape=jax.ShapeDtypeStruct((N_NODES, 256), jnp.float32),
    )(x, Wr1, b1.reshape(1, 256))


def _tc1(s1p, cnt_t, xr, Wl1, Wl2, Wr2, b2):
    n_blk = N_NODES // _ROW_BLK
    blk = lambda shape, imap: pl.BlockSpec(shape, imap)
    return pl.pallas_call(
        _tc1_body,
        grid=(n_blk,),
        in_specs=[
            blk((2, _ROW_BLK, 128), lambda i: (0, i, 0)),
            blk((_ROW_BLK, NW), lambda i: (i, 0)),
            blk((_ROW_BLK, 256), lambda i: (i, 0)),
            blk((128, 256), lambda i: (0, 0)),
            blk((256, 64), lambda i: (0, 0)),
            blk((256, 64), lambda i: (0, 0)),
            blk((1, 64), lambda i: (0, 0)),
        ],
        out_specs=[
            blk((_ROW_BLK, 64), lambda i: (i, 0)),
            blk((_ROW_BLK, 64), lambda i: (i, 0)),
        ],
        out_shape=[
            jax.ShapeDtypeStruct((N_NODES, 64), jnp.float32),
            jax.ShapeDtypeStruct((N_NODES, 64), jnp.float32),
        ],
    )(s1p, cnt_t, xr, Wl1, Wl2, Wr2, b2.reshape(1, 64))


def _tc2(s2p, cnt_t, q):
    n_blk = N_NODES // _ROW_BLK
    blk = lambda shape, imap: pl.BlockSpec(shape, imap)
    return pl.pallas_call(
        _tc2_body,
        grid=(n_blk,),
        in_specs=[
            blk((2, _ROW_BLK, 64), lambda i: (0, i, 0)),
            blk((_ROW_BLK, NW), lambda i: (i, 0)),
            blk((_ROW_BLK, 64), lambda i: (i, 0)),
        ],
        out_specs=blk((_ROW_BLK, 64), lambda i: (i, 0)),
        out_shape=jax.ShapeDtypeStruct((N_NODES, 64), jnp.float32),
    )(s2p, cnt_t, q)


def kernel(x, edge_index, Wl1, Wr1, b1, Wl2, Wr2, b2):
    src = edge_index[0].astype(jnp.int32)
    dst = edge_index[1].astype(jnp.int32)
    pad = EDGE_ROWS * CHUNK - N_EDGES
    pad_iota = np.arange(pad, dtype=np.int32)
    src2d = jnp.concatenate(
        [src, jnp.asarray(pad_iota % N_NODES)]).reshape(EDGE_ROWS, CHUNK)
    dst2d = jnp.concatenate(
        [dst, jnp.asarray(N_NODES + pad_iota % (N_PAD - N_NODES))]
    ).reshape(EDGE_ROWS, CHUNK)

    xr = _tc0(x, Wr1, b1)
    s1p, cntp = _segsum_l1(x, src2d, dst2d)
    cnt_t = jnp.transpose(cntp.reshape(NW, N_PAD)[:, :N_NODES])
    p, q = _tc1(s1p, cnt_t, xr, Wl1, Wl2, Wr2, b2)
    (s2p,) = _segsum_l2(p, src2d.reshape(EDGE_ROWS // 2, 128),
                        dst2d.reshape(EDGE_ROWS // 2, 128))
    return _tc2(s2p, cnt_t, q)
